# baseline (speedup 1.0000x reference)
import jax
import jax.numpy as jnp
from jax import lax
from jax.experimental import pallas as pl
from jax.experimental.pallas import tpu as pltpu


def _pick_bm(n, cap):
    best = None
    for bm in range(8, cap + 1, 8):
        if n % bm == 0:
            best = bm
    return best if best is not None else n


def _pick_bk(ncols_main):
    best = 128
    for bk in range(128, 2049, 128):
        if ncols_main % bk == 0:
            best = bk
    return best


def kernel(x, adj, W1, b1, W2, b2, lin_w, lin_b):
    n, nfeat = x.shape
    h1 = W1.shape[1]
    h2 = W2.shape[1]
    ncls = lin_w.shape[1]
    ncls_pad = ((ncls + 127) // 128) * 128

    bm = _pick_bm(n, 1000)
    nblk = n // bm
    ncols_main = (n // 128) * 128
    rem = n - ncols_main
    bk = _pick_bk(ncols_main)
    nk = ncols_main // bk
    gpr = 2 if nblk % 2 == 0 else 1
    bm2 = bm * gpr
    nblk2 = n // bm2

    bf16 = jnp.bfloat16

    SW = ((max(rem, 1) + 7) // 8) * 8
    n_pad = max(n, ncols_main + SW)

    strip = lax.slice(adj, (0, ncols_main), (n, n)).astype(bf16)
    if rem < SW:
        strip = jnp.pad(strip, ((0, 0), (0, SW - rem)))
    xb = x.astype(bf16)
    xp = jnp.pad(xb, ((0, n_pad - n), (0, 0))) if n_pad > n else xb
    s2_tail = n_pad - n

    b1r = b1.reshape(1, h1)
    b2r = b2.reshape(1, h2)
    w1b = W1.astype(bf16)
    w2b = W2.astype(bf16)
    lw = jnp.pad(lin_w, ((0, 0), (0, ncls_pad - ncls))).astype(bf16)
    lb = jnp.pad(lin_b, (0, ncls_pad - ncls)).reshape(1, ncls_pad)

    def body1(adj_ref, strip_ref, x_ref, w1_ref, b1_ref, w2_ref,
              s2o_ref, oacc_o_ref, q8_ref, sc_ref,
              s2_ref, oacc_ref, tacc_ref):
        m = pl.program_id(0)
        k = pl.program_id(1)
        kf = (m * bm) // bk

        t32 = adj_ref[...]
        t = t32.astype(bf16)
        krows = pl.ds(pl.multiple_of(k * bk, 128), bk)
        prod = jnp.dot(t, x_ref[krows, :], preferred_element_type=jnp.float32)

        @pl.when(k == 0)
        def _init():
            tacc_ref[...] = prod + jnp.dot(
                strip_ref[...], x_ref[pl.ds(ncols_main, SW), :],
                preferred_element_type=jnp.float32)
            oacc_ref[...] = jnp.zeros((bm, h2), jnp.float32)
            if s2_tail > 0:
                @pl.when(m == 0)
                def _zero_tail():
                    s2_ref[pl.ds(n, s2_tail), :] = jnp.zeros(
                        (s2_tail, h2), jnp.float32)

        @pl.when(k > 0)
        def _acc():
            tacc_ref[...] += prod

        @pl.when(k < kf)
        def _dual():
            oacc_ref[...] += jnp.dot(
                t, s2_ref[krows, :].astype(bf16),
                preferred_element_type=jnp.float32)

        @pl.when(k >= kf)
        def _quant():
            amax = jnp.maximum(jnp.max(jnp.abs(t32)), 1e-30)
            scale = 126.9 / amax
            q8_ref[...] = (t32 * scale + 0.5).astype(jnp.int8)
            sc_ref[...] = jnp.full((8, 128), amax * (1.0 / 126.9),
                                   jnp.float32)

        @pl.when(k == nk - 1)
        def _epilogue1():
            z = jnp.dot(tacc_ref[...].astype(bf16), w1_ref[...],
                        preferred_element_type=jnp.float32) + b1_ref[...]
            h = jnp.maximum(z, 0.1 * z)
            s2v = jnp.dot(h.astype(bf16), w2_ref[...],
                          preferred_element_type=jnp.float32)
            s2_ref[pl.ds(pl.multiple_of(m * bm, 8), bm), :] = s2v
            s2o_ref[...] = s2v.astype(bf16)
            oacc_o_ref[...] = oacc_ref[...]

    def q8_index1(m, k):
        return (m, jnp.minimum(jnp.maximum(k, (m * bm) // bk), nk - 1))

    s2_full, oacc_part, q8, scales = pl.pallas_call(
        body1,
        grid=(nblk, nk),
        in_specs=[
            pl.BlockSpec((bm, bk), lambda m, k: (m, k)),
            pl.BlockSpec((bm, SW), lambda m, k: (m, 0)),
            pl.BlockSpec((n_pad, nfeat), lambda m, k: (0, 0)),
            pl.BlockSpec((nfeat, h1), lambda m, k: (0, 0)),
            pl.BlockSpec((1, h1), lambda m, k: (0, 0)),
            pl.BlockSpec((h1, h2), lambda m, k: (0, 0)),
        ],
        out_specs=[
            pl.BlockSpec((bm, h2), lambda m, k: (m, 0)),
            pl.BlockSpec((bm, h2), lambda m, k: (m, 0)),
            pl.BlockSpec((bm, bk), q8_index1),
            pl.BlockSpec((8, 128), lambda m, k: (m * nk + k, 0)),
        ],
        out_shape=[
            jax.ShapeDtypeStruct((n_pad, h2), bf16),
            jax.ShapeDtypeStruct((n, h2), jnp.float32),
            jax.ShapeDtypeStruct((n, ncols_main), jnp.int8),
            jax.ShapeDtypeStruct((nblk * nk * 8, 128), jnp.float32),
        ],
        scratch_shapes=[
            pltpu.VMEM((n_pad, h2), jnp.float32),
            pltpu.VMEM((bm, h2), jnp.float32),
            pltpu.VMEM((bm, nfeat), jnp.float32),
        ],
        compiler_params=pltpu.CompilerParams(
            dimension_semantics=("arbitrary", "arbitrary"),
            vmem_limit_bytes=64 * 1024 * 1024,
        ),
    )(adj, strip, xp, w1b, b1r, w2b)

    def body2(q8_ref, scs_ref, s2_ref, oin_ref, strip_ref, b2_ref,
              lw_ref, lb_ref, out_ref, oacc_ref):
        g = pl.program_id(0)
        k = pl.program_id(1)
        krows = pl.ds(pl.multiple_of(k * bk, 128), bk)

        @pl.when(k == 0)
        def _init():
            oacc_ref[...] = oin_ref[...]

        kf_first = (g * gpr * bm) // bk

        @pl.when(k >= kf_first)
        def _upper():
            q = q8_ref[...].astype(bf16)
            d = jnp.dot(q, s2_ref[krows, :],
                        preferred_element_type=jnp.float32)
            for j in range(gpr):
                a = g * gpr + j
                kf_j = (a * bm) // bk if gpr == 1 else None
                rows = slice(j * bm, (j + 1) * bm)
                sv = scs_ref[pl.ds(j * 8, 8), :][0:1, :]
                if gpr == 1:
                    oacc_ref[rows, :] += d[rows, :] * sv
                else:
                    @pl.when(k >= (a * bm) // bk)
                    def _half(j=j, rows=rows, sv=sv):
                        oacc_ref[rows, :] += d[rows, :] * sv

        @pl.when(k == nk - 1)
        def _epilogue2():
            z = oacc_ref[...] + jnp.dot(
                strip_ref[...], s2_ref[pl.ds(ncols_main, SW), :],
                preferred_element_type=jnp.float32) + b2_ref[...]
            h = jnp.maximum(z, 0.1 * z)
            out_ref[...] = jnp.dot(
                h.astype(bf16), lw_ref[...],
                preferred_element_type=jnp.float32) + lb_ref[...]

    def q8_index2(g, k):
        kf = (g * gpr * bm) // bk
        return (g, jnp.minimum(jnp.maximum(k, kf), nk - 1))

    def sc_index2(g, k):
        return (g * nk + k, 0)

    sc3 = scales.reshape(nblk, nk, 8, 128)
    sc3 = sc3.reshape(nblk2, gpr, nk, 8, 128).transpose(0, 2, 1, 3, 4)
    sc2 = sc3.reshape(nblk2 * nk * gpr * 8, 128)

    out = pl.pallas_call(
        body2,
        grid=(nblk2, nk),
        in_specs=[
            pl.BlockSpec((bm2, bk), q8_index2),
            pl.BlockSpec((8 * gpr, 128), lambda g, k: (g * nk + k, 0)),
            pl.BlockSpec((n_pad, h2), lambda g, k: (0, 0)),
            pl.BlockSpec((bm2, h2), lambda g, k: (g, 0)),
            pl.BlockSpec((bm2, SW), lambda g, k: (g, 0)),
            pl.BlockSpec((1, h2), lambda g, k: (0, 0)),
            pl.BlockSpec((h2, ncls_pad), lambda g, k: (0, 0)),
            pl.BlockSpec((1, ncls_pad), lambda g, k: (0, 0)),
        ],
        out_specs=pl.BlockSpec((bm2, ncls_pad), lambda g, k: (g, 0)),
        out_shape=jax.ShapeDtypeStruct((n, ncls_pad), jnp.float32),
        scratch_shapes=[
            pltpu.VMEM((bm2, h2), jnp.float32),
        ],
        compiler_params=pltpu.CompilerParams(
            dimension_semantics=("arbitrary", "arbitrary"),
            vmem_limit_bytes=64 * 1024 * 1024,
        ),
    )(q8, sc2, s2_full, oacc_part, strip, b2r, lw, lb)

    return out[:, :ncls]


# speedup vs baseline: 1.2329x; 1.2329x over previous
import jax
import jax.numpy as jnp
from jax import lax
from jax.experimental import pallas as pl
from jax.experimental.pallas import tpu as pltpu


def _pick_bm(n):
    best = None
    for bm in range(8, 2001, 8):
        if n % bm == 0:
            best = bm
    return best if best is not None else n


def _pick_bk(ncols_main):
    best = 128
    for bk in range(128, 2049, 128):
        if ncols_main % bk == 0:
            best = bk
    return best


def kernel(x, adj, W1, b1, W2, b2, lin_w, lin_b):
    n, nfeat = x.shape
    h1 = W1.shape[1]
    h2 = W2.shape[1]
    ncls = lin_w.shape[1]
    ncls_pad = ((ncls + 127) // 128) * 128

    bm = _pick_bm(n)
    nblk = n // bm
    ncols_main = (n // 128) * 128
    rem = n - ncols_main
    bk = _pick_bk(ncols_main)
    nk = ncols_main // bk

    bf16 = jnp.bfloat16

    SW = ((max(rem, 1) + 7) // 8) * 8
    n_pad = max(n, ncols_main + SW)

    strip = lax.slice(adj, (0, ncols_main), (n, n)).astype(bf16)
    if rem < SW:
        strip = jnp.pad(strip, ((0, 0), (0, SW - rem)))
    xb = x.astype(bf16)
    xp = jnp.pad(xb, ((0, n_pad - n), (0, 0))) if n_pad > n else xb
    s2_tail = n_pad - n

    b1r = b1.reshape(1, h1)
    b2r = b2.reshape(1, h2)
    w1b = W1.astype(bf16)
    w2b = W2.astype(bf16)
    lw = jnp.pad(lin_w, ((0, 0), (0, ncls_pad - ncls))).astype(bf16)
    lb = jnp.pad(lin_b, (0, ncls_pad - ncls)).reshape(1, ncls_pad)

    def body(adj_ref, strip_ref, x_ref, w1_ref, b1_ref, w2_ref, b2_ref,
             lw_ref, lb_ref, out_ref, s2_ref, oacc_ref, tacc_ref):
        p = pl.program_id(0)
        m = pl.program_id(1)
        k = pl.program_id(2)

        t = adj_ref[...].astype(bf16)
        mrows = pl.ds(m * bm, bm)
        krows = pl.ds(k * bk, bk)
        kf = (m * bm) // bk

        @pl.when(p == 0)
        def _phase0():
            prod = jnp.dot(t, x_ref[krows, :],
                           preferred_element_type=jnp.float32)

            @pl.when(k == 0)
            def _init():
                tacc_ref[...] = prod + jnp.dot(
                    strip_ref[...], x_ref[pl.ds(ncols_main, SW), :],
                    preferred_element_type=jnp.float32)
                oacc_ref[mrows, :] = jnp.zeros((bm, h2), jnp.float32)
                if s2_tail > 0:
                    @pl.when(m == 0)
                    def _zero_tail():
                        s2_ref[pl.ds(n, s2_tail), :] = jnp.zeros(
                            (s2_tail, h2), bf16)

            @pl.when(k > 0)
            def _acc():
                tacc_ref[...] += prod

            @pl.when(k < kf)
            def _dual():
                oacc_ref[mrows, :] += jnp.dot(
                    t, s2_ref[krows, :], preferred_element_type=jnp.float32)

            @pl.when(k == nk - 1)
            def _epilogue1():
                z = jnp.dot(tacc_ref[...].astype(bf16), w1_ref[...],
                            preferred_element_type=jnp.float32) + b1_ref[...]
                h = jnp.maximum(z, 0.1 * z)
                s2_ref[mrows, :] = jnp.dot(
                    h.astype(bf16), w2_ref[...],
                    preferred_element_type=jnp.float32).astype(bf16)

        @pl.when(p == 1)
        def _phase1():
            @pl.when(k >= kf)
            def _upper():
                oacc_ref[mrows, :] += jnp.dot(
                    t, s2_ref[krows, :], preferred_element_type=jnp.float32)

            @pl.when(k == nk - 1)
            def _epilogue2():
                z = oacc_ref[mrows, :] + jnp.dot(
                    strip_ref[...], s2_ref[pl.ds(ncols_main, SW), :],
                    preferred_element_type=jnp.float32) + b2_ref[...]
                h = jnp.maximum(z, 0.1 * z)
                o = jnp.dot(
                    h.astype(bf16), lw_ref[...],
                    preferred_element_type=jnp.float32) + lb_ref[...]
                out_ref[...] = o[:, :ncls]

    def adj_index(p, m, k):
        kf = (m * bm) // bk
        return (m, jnp.where(p == 0, k, jnp.maximum(k, kf)))

    out = pl.pallas_call(
        body,
        grid=(2, nblk, nk),
        in_specs=[
            pl.BlockSpec((bm, bk), adj_index),
            pl.BlockSpec((bm, SW), lambda p, m, k: (m, 0)),
            pl.BlockSpec((n_pad, nfeat), lambda p, m, k: (0, 0)),
            pl.BlockSpec((nfeat, h1), lambda p, m, k: (0, 0)),
            pl.BlockSpec((1, h1), lambda p, m, k: (0, 0)),
            pl.BlockSpec((h1, h2), lambda p, m, k: (0, 0)),
            pl.BlockSpec((1, h2), lambda p, m, k: (0, 0)),
            pl.BlockSpec((h2, ncls_pad), lambda p, m, k: (0, 0)),
            pl.BlockSpec((1, ncls_pad), lambda p, m, k: (0, 0)),
        ],
        out_specs=pl.BlockSpec((bm, ncls), lambda p, m, k: (p * m, 0)),
        out_shape=jax.ShapeDtypeStruct((n, ncls), jnp.float32),
        scratch_shapes=[
            pltpu.VMEM((n_pad, h2), bf16),
            pltpu.VMEM((n, h2), jnp.float32),
            pltpu.VMEM((bm, nfeat), jnp.float32),
        ],
        compiler_params=pltpu.CompilerParams(
            dimension_semantics=("arbitrary", "arbitrary", "arbitrary"),
            vmem_limit_bytes=64 * 1024 * 1024,
        ),
    )(adj, strip, xp, w1b, b1r, w2b, b2r, lw, lb)

    return out


# speedup vs baseline: 1.2335x; 1.0005x over previous
import jax
import jax.numpy as jnp
from jax import lax
from jax.experimental import pallas as pl
from jax.experimental.pallas import tpu as pltpu


def _pick_bm(n):
    best = None
    for bm in range(8, 2001, 8):
        if n % bm == 0:
            best = bm
    return best if best is not None else n


def _pick_bk(ncols_main):
    best = 128
    for bk in range(128, 2049, 128):
        if ncols_main % bk == 0:
            best = bk
    return best


def kernel(x, adj, W1, b1, W2, b2, lin_w, lin_b):
    n, nfeat = x.shape
    h1 = W1.shape[1]
    h2 = W2.shape[1]
    ncls = lin_w.shape[1]
    ncls_pad = ((ncls + 127) // 128) * 128

    bm = _pick_bm(n)
    nblk = n // bm
    ncols_main = (n // 128) * 128
    rem = n - ncols_main
    bk = _pick_bk(ncols_main)
    nk = ncols_main // bk

    bf16 = jnp.bfloat16

    SW = ((max(rem, 1) + 7) // 8) * 8
    n_pad = max(n, ncols_main + SW)

    strip = lax.slice(adj, (0, ncols_main), (n, n)).astype(bf16)
    if rem < SW:
        strip = jnp.pad(strip, ((0, 0), (0, SW - rem)))
    xb = x.astype(bf16)
    xp = jnp.pad(xb, ((0, n_pad - n), (0, 0))) if n_pad > n else xb
    s2_tail = n_pad - n

    b1r = b1.reshape(1, h1)
    b2r = b2.reshape(1, h2)
    w1b = W1.astype(bf16)
    w2b = W2.astype(bf16)
    lw = jnp.pad(lin_w, ((0, 0), (0, ncls_pad - ncls))).astype(bf16)
    lb = jnp.pad(lin_b, (0, ncls_pad - ncls)).reshape(1, ncls_pad)

    def body(adj_ref, strip_ref, x_ref, w1_ref, b1_ref, w2_ref, b2_ref,
             lw_ref, lb_ref, out_ref, s2_ref, oacc_ref, tacc_ref):
        p = pl.program_id(0)
        m = pl.program_id(1)
        k = pl.program_id(2)

        t = adj_ref[...].astype(bf16)
        mrows = pl.ds(m * bm, bm)
        krows = pl.ds(k * bk, bk)
        kf = (m * bm) // bk

        @pl.when(p == 0)
        def _phase0():
            prod = jnp.dot(t, x_ref[krows, :],
                           preferred_element_type=jnp.float32)

            @pl.when(k == 0)
            def _init():
                tacc_ref[...] = prod + jnp.dot(
                    strip_ref[...], x_ref[pl.ds(ncols_main, SW), :],
                    preferred_element_type=jnp.float32)
                oacc_ref[mrows, :] = jnp.zeros((bm, h2), jnp.float32)
                if s2_tail > 0:
                    @pl.when(m == 0)
                    def _zero_tail():
                        s2_ref[pl.ds(n, s2_tail), :] = jnp.zeros(
                            (s2_tail, h2), bf16)

            @pl.when(k > 0)
            def _acc():
                tacc_ref[...] += prod

            @pl.when(k < kf)
            def _dual():
                oacc_ref[mrows, :] += jnp.dot(
                    t, s2_ref[krows, :], preferred_element_type=jnp.float32)

            @pl.when(k == nk - 1)
            def _epilogue1():
                z = jnp.dot(tacc_ref[...].astype(bf16), w1_ref[...],
                            preferred_element_type=jnp.float32) + b1_ref[...]
                h = jnp.maximum(z, 0.1 * z)
                s2_ref[mrows, :] = jnp.dot(
                    h.astype(bf16), w2_ref[...],
                    preferred_element_type=jnp.float32).astype(bf16)

        @pl.when(p == 1)
        def _phase1():
            @pl.when(k >= kf)
            def _upper():
                oacc_ref[mrows, :] += jnp.dot(
                    t, s2_ref[krows, :], preferred_element_type=jnp.float32)

            @pl.when(k == nk - 1)
            def _epilogue2():
                z = oacc_ref[mrows, :] + jnp.dot(
                    strip_ref[...], s2_ref[pl.ds(ncols_main, SW), :],
                    preferred_element_type=jnp.float32) + b2_ref[...]
                h = jnp.maximum(z, 0.1 * z)
                out_ref[...] = jnp.dot(
                    h.astype(bf16), lw_ref[...],
                    preferred_element_type=jnp.float32) + lb_ref[...]

    def adj_index(p, m, k):
        kf = (m * bm) // bk
        return (m, jnp.where(p == 0, k, jnp.maximum(k, kf)))

    out = pl.pallas_call(
        body,
        grid=(2, nblk, nk),
        in_specs=[
            pl.BlockSpec((bm, bk), adj_index),
            pl.BlockSpec((bm, SW), lambda p, m, k: (m, 0)),
            pl.BlockSpec((n_pad, nfeat), lambda p, m, k: (0, 0)),
            pl.BlockSpec((nfeat, h1), lambda p, m, k: (0, 0)),
            pl.BlockSpec((1, h1), lambda p, m, k: (0, 0)),
            pl.BlockSpec((h1, h2), lambda p, m, k: (0, 0)),
            pl.BlockSpec((1, h2), lambda p, m, k: (0, 0)),
            pl.BlockSpec((h2, ncls_pad), lambda p, m, k: (0, 0)),
            pl.BlockSpec((1, ncls_pad), lambda p, m, k: (0, 0)),
        ],
        out_specs=pl.BlockSpec((bm, ncls_pad), lambda p, m, k: (p * m, 0)),
        out_shape=jax.ShapeDtypeStruct((n, ncls_pad), jnp.float32),
        scratch_shapes=[
            pltpu.VMEM((n_pad, h2), bf16),
            pltpu.VMEM((n, h2), jnp.float32),
            pltpu.VMEM((bm, nfeat), jnp.float32),
        ],
        compiler_params=pltpu.CompilerParams(
            dimension_semantics=("arbitrary", "arbitrary", "arbitrary"),
            vmem_limit_bytes=64 * 1024 * 1024,
        ),
    )(adj, strip, xp, w1b, b1r, w2b, b2r, lw, lb)

    return out[:, :ncls]


# speedup vs baseline: 1.2492x; 1.0128x over previous
import jax
import jax.numpy as jnp
from jax import lax
from jax.experimental import pallas as pl
from jax.experimental.pallas import tpu as pltpu


def _pick_bm(n):
    best = None
    for bm in range(8, 2001, 8):
        if n % bm == 0:
            best = bm
    return best if best is not None else n


def _pick_bk(ncols_main):
    best = 128
    for bk in range(128, 2049, 128):
        if ncols_main % bk == 0:
            best = bk
    return best


def kernel(x, adj, W1, b1, W2, b2, lin_w, lin_b):
    n, nfeat = x.shape
    h1 = W1.shape[1]
    h2 = W2.shape[1]
    ncls = lin_w.shape[1]
    ncls_pad = ((ncls + 127) // 128) * 128

    bm = _pick_bm(n)
    nblk = n // bm
    ncols_main = (n // 128) * 128
    rem = n - ncols_main
    bk = _pick_bk(ncols_main)
    nk = ncols_main // bk

    bf16 = jnp.bfloat16

    SW = ((max(rem, 1) + 7) // 8) * 8
    n_pad = max(n, ncols_main + SW)

    strip = lax.slice(adj, (0, ncols_main), (n, n)).astype(bf16)
    if rem < SW:
        strip = jnp.pad(strip, ((0, 0), (0, SW - rem)))
    xb = x.astype(bf16)
    xp = jnp.pad(xb, ((0, n_pad - n), (0, 0))) if n_pad > n else xb
    s2_tail = n_pad - n

    b1r = b1.reshape(1, h1)
    b2r = b2.reshape(1, h2)
    w1b = W1.astype(bf16)
    w2b = W2.astype(bf16)
    lw = jnp.pad(lin_w, ((0, 0), (0, ncls_pad - ncls))).astype(bf16)
    lb = jnp.pad(lin_b, (0, ncls_pad - ncls)).reshape(1, ncls_pad)

    def body(adj_ref, strip_ref, x_ref, w1_ref, b1_ref, w2_ref, b2_ref,
             lw_ref, lb_ref, out_ref, s2_ref, oacc_ref, tacc_ref):
        p = pl.program_id(0)
        m = pl.program_id(1)
        k = pl.program_id(2)

        t = adj_ref[...].astype(bf16)
        mrows = pl.ds(m * bm, bm)
        krows = pl.ds(k * bk, bk)
        kf = (m * bm) // bk

        @pl.when(p == 0)
        def _phase0():
            prod = jnp.dot(t, x_ref[krows, :],
                           preferred_element_type=jnp.float32)

            @pl.when(k == 0)
            def _init():
                tacc_ref[...] = prod + jnp.dot(
                    strip_ref[...], x_ref[pl.ds(ncols_main, SW), :],
                    preferred_element_type=jnp.float32)
                oacc_ref[mrows, :] = jnp.zeros((bm, h2), jnp.float32)
                if s2_tail > 0:
                    @pl.when(m == 0)
                    def _zero_tail():
                        s2_ref[pl.ds(n, s2_tail), :] = jnp.zeros(
                            (s2_tail, h2), bf16)

            @pl.when(k > 0)
            def _acc():
                tacc_ref[...] += prod

            @pl.when(k < kf)
            def _dual():
                oacc_ref[mrows, :] += jnp.dot(
                    t, s2_ref[krows, :], preferred_element_type=jnp.float32)

            @pl.when(k == nk - 1)
            def _epilogue1():
                z = jnp.dot(tacc_ref[...].astype(bf16), w1_ref[...],
                            preferred_element_type=jnp.float32) + b1_ref[...]
                h = jnp.maximum(z, 0.1 * z)
                s2_ref[mrows, :] = jnp.dot(
                    h.astype(bf16), w2_ref[...],
                    preferred_element_type=jnp.float32).astype(bf16)

        @pl.when(p == 1)
        def _phase1():
            @pl.when(k >= kf)
            def _upper():
                oacc_ref[mrows, :] += jnp.dot(
                    t, s2_ref[krows, :], preferred_element_type=jnp.float32)

            @pl.when(k == nk - 1)
            def _epilogue2():
                z = oacc_ref[mrows, :] + jnp.dot(
                    strip_ref[...], s2_ref[pl.ds(ncols_main, SW), :],
                    preferred_element_type=jnp.float32) + b2_ref[...]
                h = jnp.maximum(z, 0.1 * z)
                out_ref[...] = jnp.dot(
                    h.astype(bf16), lw_ref[...],
                    preferred_element_type=jnp.float32) + lb_ref[...]

    def adj_index(p, m, k):
        kf = (m * bm) // bk
        return (m, jnp.where(p == 0, k, jnp.maximum(k, kf)))

    out = pl.pallas_call(
        body,
        grid=(2, nblk, nk),
        in_specs=[
            pl.BlockSpec((bm, bk), adj_index),
            pl.BlockSpec((bm, SW), lambda p, m, k: (m, 0)),
            pl.BlockSpec((n_pad, nfeat), lambda p, m, k: (0, 0)),
            pl.BlockSpec((nfeat, h1), lambda p, m, k: (0, 0)),
            pl.BlockSpec((1, h1), lambda p, m, k: (0, 0)),
            pl.BlockSpec((h1, h2), lambda p, m, k: (0, 0)),
            pl.BlockSpec((1, h2), lambda p, m, k: (0, 0)),
            pl.BlockSpec((h2, ncls_pad), lambda p, m, k: (0, 0)),
            pl.BlockSpec((1, ncls_pad), lambda p, m, k: (0, 0)),
        ],
        out_specs=pl.BlockSpec((bm, ncls_pad), lambda p, m, k: (p * m, 0)),
        out_shape=jax.ShapeDtypeStruct((n, ncls_pad), jnp.float32),
        scratch_shapes=[
            pltpu.VMEM((n_pad, h2), bf16),
            pltpu.VMEM((n, h2), jnp.float32),
            pltpu.VMEM((bm, nfeat), jnp.float32),
        ],
        compiler_params=pltpu.CompilerParams(
            dimension_semantics=("arbitrary", "arbitrary", "arbitrary"),
            vmem_limit_bytes=64 * 1024 * 1024,
        ),
    )(adj, strip, xp, W1, b1r, W2, b2r, lw, lb)

    return out[:, :ncls]


# speedup vs baseline: 1.2500x; 1.0007x over previous
import jax
import jax.numpy as jnp
from jax import lax
from jax.experimental import pallas as pl
from jax.experimental.pallas import tpu as pltpu


def _pick_bm(n):
    best = None
    for bm in range(8, 2001, 8):
        if n % bm == 0:
            best = bm
    return best if best is not None else n


def _pick_bk(ncols_main):
    best = 128
    for bk in range(128, 2049, 128):
        if ncols_main % bk == 0:
            best = bk
    return best


def kernel(x, adj, W1, b1, W2, b2, lin_w, lin_b):
    n, nfeat = x.shape
    h1 = W1.shape[1]
    h2 = W2.shape[1]
    ncls = lin_w.shape[1]
    ncls_pad = ((ncls + 127) // 128) * 128

    bm = _pick_bm(n)
    nblk = n // bm
    ncols_main = (n // 128) * 128
    rem = n - ncols_main
    bk = _pick_bk(ncols_main)
    nk = ncols_main // bk

    bf16 = jnp.bfloat16

    SW = ((max(rem, 1) + 7) // 8) * 8
    n_pad = max(n, ncols_main + SW)

    strip = lax.slice(adj, (0, ncols_main), (n, n)).astype(bf16)
    if rem < SW:
        strip = jnp.pad(strip, ((0, 0), (0, SW - rem)))
    xb = x.astype(bf16)
    xp = jnp.pad(xb, ((0, n_pad - n), (0, 0))) if n_pad > n else xb
    s2_tail = n_pad - n

    b1r = b1.reshape(1, h1)
    b2r = b2.reshape(1, h2)
    lw = jnp.pad(lin_w, ((0, 0), (0, ncls_pad - ncls))).astype(bf16)
    lb = jnp.pad(lin_b, (0, ncls_pad - ncls)).reshape(1, ncls_pad)

    def body(adj_ref, strip_ref, x_ref, w1_ref, b1_ref, w2_ref, b2_ref,
             lw_ref, lb_ref, out_ref, s2_ref, oacc_ref, tacc_ref):
        p = pl.program_id(0)
        m = pl.program_id(1)
        k = pl.program_id(2)

        t = adj_ref[...].astype(bf16)
        mrows = pl.ds(m * bm, bm)
        krows = pl.ds(k * bk, bk)
        kf = (m * bm) // bk

        @pl.when(p == 0)
        def _phase0():
            prod = jnp.dot(t, x_ref[krows, :],
                           preferred_element_type=jnp.float32)

            @pl.when(k == 0)
            def _init():
                tacc_ref[...] = prod + jnp.dot(
                    strip_ref[...], x_ref[pl.ds(ncols_main, SW), :],
                    preferred_element_type=jnp.float32)
                oacc_ref[mrows, :] = jnp.zeros((bm, h2), jnp.float32)
                if s2_tail > 0:
                    @pl.when(m == 0)
                    def _zero_tail():
                        s2_ref[pl.ds(n, s2_tail), :] = jnp.zeros(
                            (s2_tail, h2), bf16)

            @pl.when(k > 0)
            def _acc():
                tacc_ref[...] += prod

            @pl.when(k < kf)
            def _dual():
                oacc_ref[mrows, :] += jnp.dot(
                    t, s2_ref[krows, :], preferred_element_type=jnp.float32)

            @pl.when(k == nk - 1)
            def _epilogue1():
                z = jnp.dot(tacc_ref[...].astype(bf16), w1_ref[...],
                            preferred_element_type=jnp.float32) + b1_ref[...]
                h = jnp.maximum(z, 0.1 * z)
                s2_ref[mrows, :] = jnp.dot(
                    h.astype(bf16), w2_ref[...],
                    preferred_element_type=jnp.float32).astype(bf16)

        @pl.when(p == 1)
        def _phase1():
            @pl.when(k >= kf)
            def _upper():
                oacc_ref[mrows, :] += jnp.dot(
                    t, s2_ref[krows, :], preferred_element_type=jnp.float32)

            @pl.when(k == nk - 1)
            def _epilogue2():
                z = oacc_ref[mrows, :] + jnp.dot(
                    strip_ref[...], s2_ref[pl.ds(ncols_main, SW), :],
                    preferred_element_type=jnp.float32) + b2_ref[...]
                h = jnp.maximum(z, 0.1 * z)
                out_ref[...] = jnp.dot(
                    h.astype(bf16), lw_ref[...],
                    preferred_element_type=jnp.float32) + lb_ref[...]

    def adj_index(p, m, k):
        kf = (m * bm) // bk
        return (m, jnp.where(p == 0, k, jnp.maximum(k, kf)))

    out = pl.pallas_call(
        body,
        grid=(2, nblk, nk),
        in_specs=[
            pl.BlockSpec((bm, bk), adj_index),
            pl.BlockSpec((bm, SW), lambda p, m, k: (m, 0)),
            pl.BlockSpec((n_pad, nfeat), lambda p, m, k: (0, 0)),
            pl.BlockSpec((nfeat, h1), lambda p, m, k: (0, 0)),
            pl.BlockSpec((1, h1), lambda p, m, k: (0, 0)),
            pl.BlockSpec((h1, h2), lambda p, m, k: (0, 0)),
            pl.BlockSpec((1, h2), lambda p, m, k: (0, 0)),
            pl.BlockSpec((h2, ncls_pad), lambda p, m, k: (0, 0)),
            pl.BlockSpec((1, ncls_pad), lambda p, m, k: (0, 0)),
        ],
        out_specs=pl.BlockSpec((bm, ncls_pad), lambda p, m, k: (p * m, 0)),
        out_shape=jax.ShapeDtypeStruct((n, ncls_pad), jnp.float32),
        scratch_shapes=[
            pltpu.VMEM((n_pad, h2), bf16),
            pltpu.VMEM((n, h2), jnp.float32),
            pltpu.VMEM((bm, nfeat), jnp.float32),
        ],
        compiler_params=pltpu.CompilerParams(
            dimension_semantics=("arbitrary", "arbitrary", "arbitrary"),
            vmem_limit_bytes=64 * 1024 * 1024,
        ),
    )(adj, strip, xp, W1, b1r, W2, b2r, lw, lb)

    return out[:, :ncls]
